# baseline (speedup 1.0000x reference)
import functools

import jax
import jax.numpy as jnp
from jax import lax
from jax.experimental import pallas as pl
from jax.experimental.pallas import tpu as pltpu
from jax.experimental.pallas import tpu_sc as plsc

LABP = 16


def _proj_body(table_ref, w_ref, b_ref, out_ref):
    out_ref[...] = (
        jnp.dot(table_ref[...], w_ref[...], preferred_element_type=jnp.float32)
        + b_ref[...]
    )


def _make_gather(vp, n, n_per_w, nc, lab):
    mesh = plsc.VectorSubcoreMesh(core_axis_name="c", subcore_axis_name="s")

    @functools.partial(
        pl.kernel,
        mesh=mesh,
        out_type=jax.ShapeDtypeStruct((n * lab,), jnp.float32),
        scratch_types=[
            pltpu.VMEM((vp * LABP,), jnp.float32),
            pltpu.VMEM((n_per_w,), jnp.int32),
            pltpu.VMEM((n_per_w * lab,), jnp.float32),
        ],
        compiler_params=pltpu.CompilerParams(
            use_tc_tiling_on_sc=False,
            needs_layout_passes=False,
            disable_bounds_checks=True,
        ),
    )
    def gather_k(ptab_hbm, idx_hbm, out_hbm, ptab_v, idx_v, comp_v):
        wid = lax.axis_index("s") * nc + lax.axis_index("c")
        base = wid * n_per_w
        pltpu.sync_copy(ptab_hbm, ptab_v)
        pltpu.sync_copy(idx_hbm.at[pl.ds(base, n_per_w)], idx_v)

        lane = lax.iota(jnp.int32, 16)
        rowpats = [(lane + 16 * k) // lab for k in range(5)]
        colpats = [(lane + 16 * k) % lab for k in range(5)]

        @plsc.parallel_loop(0, n_per_w, 8, unroll=8)
        def group(t0):
            j0 = t0 * lab
            for k in range(5):
                tok = plsc.load_gather(idx_v, [t0 + rowpats[k]])
                vals = plsc.load_gather(ptab_v, [tok * LABP + colpats[k]])
                comp_v[pl.ds(j0 + 16 * k, 16)] = vals

        pltpu.sync_copy(comp_v, out_hbm.at[pl.ds(base * lab, n_per_w * lab)])

    return gather_k


def kernel(text, table, W, b):
    B, L = text.shape
    V, E = table.shape
    LAB = W.shape[1]

    w_pad = jnp.zeros((E, LABP), jnp.float32).at[:, :LAB].set(W)
    b_pad = jnp.zeros((1, LABP), jnp.float32).at[0, :LAB].set(b)
    proj = pl.pallas_call(
        _proj_body,
        out_shape=jax.ShapeDtypeStruct((V, LABP), jnp.float32),
    )(table, w_pad, b_pad)

    info = plsc.get_sparse_core_info()
    nw = info.num_cores * info.num_subcores
    n = B * L
    n_per_w = n // nw

    idx = text.reshape(n).astype(jnp.int32)
    rows = _make_gather(V, n, n_per_w, info.num_cores, LAB)(
        proj.reshape(V * LABP), idx
    )
    return rows.reshape(B, L, LAB)


# speedup vs baseline: 2.1915x; 2.1915x over previous
import functools

import jax
import jax.numpy as jnp
from jax import lax
from jax.experimental import pallas as pl
from jax.experimental.pallas import tpu as pltpu
from jax.experimental.pallas import tpu_sc as plsc

LABP = 16


def _proj_body(table_ref, w_ref, b_ref, out_ref):
    out_ref[...] = (
        jnp.dot(table_ref[...], w_ref[...], preferred_element_type=jnp.float32)
        + b_ref[...]
    )


def _make_gather(vp, n, n_per_w, nc, lab):
    mesh = plsc.VectorSubcoreMesh(core_axis_name="c", subcore_axis_name="s")

    @functools.partial(
        pl.kernel,
        mesh=mesh,
        out_type=jax.ShapeDtypeStruct((lab, n), jnp.float32),
        scratch_types=[
            pltpu.VMEM((vp * LABP,), jnp.float32),
            pltpu.VMEM((n_per_w,), jnp.int32),
            pltpu.VMEM((lab, n_per_w), jnp.float32),
        ],
        compiler_params=pltpu.CompilerParams(
            use_tc_tiling_on_sc=False,
            needs_layout_passes=False,
            disable_bounds_checks=True,
        ),
    )
    def gather_k(ptab_hbm, idx_hbm, out_hbm, ptab_v, idx_v, comp_v):
        wid = lax.axis_index("s") * nc + lax.axis_index("c")
        base = wid * n_per_w
        pltpu.sync_copy(ptab_hbm, ptab_v)
        pltpu.sync_copy(idx_hbm.at[pl.ds(base, n_per_w)], idx_v)

        @plsc.parallel_loop(0, n_per_w, 16, unroll=8)
        def group(i0):
            addr0 = idx_v[pl.ds(i0, 16)] * LABP
            for l in range(lab):
                comp_v[l, pl.ds(i0, 16)] = plsc.load_gather(
                    ptab_v, [addr0 + l]
                )
        pltpu.sync_copy(comp_v, out_hbm.at[:, pl.ds(base, n_per_w)])

    return gather_k


def kernel(text, table, W, b):
    B, L = text.shape
    V, E = table.shape
    LAB = W.shape[1]

    w_pad = jnp.zeros((E, LABP), jnp.float32).at[:, :LAB].set(W)
    b_pad = jnp.zeros((1, LABP), jnp.float32).at[0, :LAB].set(b)
    proj = pl.pallas_call(
        _proj_body,
        out_shape=jax.ShapeDtypeStruct((V, LABP), jnp.float32),
    )(table, w_pad, b_pad)

    info = plsc.get_sparse_core_info()
    nw = info.num_cores * info.num_subcores
    n = B * L
    n_per_w = n // nw

    idx = text.reshape(n).astype(jnp.int32)
    rows_t = _make_gather(V, n, n_per_w, info.num_cores, LAB)(
        proj.reshape(V * LABP), idx
    )
    return jnp.transpose(rows_t.reshape(LAB, B, L), (1, 2, 0))
